# baseline (speedup 1.0000x reference)
import jax
import jax.numpy as jnp
from jax.experimental import pallas as pl
from jax.experimental.pallas import tpu as pltpu

_SPLIT = 2


def _trim_passthrough_kernel(x_ref, v_ref, m_ref, xo_ref, vo_ref, mo_ref):
    xo_ref[...] = x_ref[...]

    @pl.when(pl.program_id(0) == 0)
    def _():
        vo_ref[...] = v_ref[...]
        mo_ref[...] = m_ref[...] != 0.0


def kernel(x, v, mask):
    B, H, L = x.shape
    h = H // _SPLIT
    xspec = pl.BlockSpec((1, h, L), lambda i: (i // _SPLIT, i % _SPLIT, 0))
    out = pl.pallas_call(
        _trim_passthrough_kernel,
        grid=(B * _SPLIT,),
        in_specs=[
            xspec,
            pl.BlockSpec(v.shape, lambda i: (0, 0, 0)),
            pl.BlockSpec(mask.shape, lambda i: (0, 0, 0)),
        ],
        out_specs=[
            xspec,
            pl.BlockSpec(v.shape, lambda i: (0, 0, 0)),
            pl.BlockSpec(mask.shape, lambda i: (0, 0, 0)),
        ],
        out_shape=[
            jax.ShapeDtypeStruct(x.shape, x.dtype),
            jax.ShapeDtypeStruct(v.shape, v.dtype),
            jax.ShapeDtypeStruct(mask.shape, jnp.bool_),
        ],
    )(x, v, mask)
    return (out[0], out[1], out[2])


# speedup vs baseline: 1.2705x; 1.2705x over previous
import jax
import jax.numpy as jnp
from jax.experimental import pallas as pl
from jax.experimental.pallas import tpu as pltpu

_NBUF = 4


def _trim_passthrough_kernel(x_hbm, v_ref, m_ref, xo_hbm, vo_ref, mo_ref,
                             buf, sem_in, sem_out):
    nchunk = x_hbm.shape[0]

    def in_copy(i):
        return pltpu.make_async_copy(x_hbm.at[i], buf.at[i % _NBUF],
                                     sem_in.at[i % _NBUF])

    def out_copy(i):
        return pltpu.make_async_copy(buf.at[i % _NBUF], xo_hbm.at[i],
                                     sem_out.at[i % _NBUF])

    for i in range(_NBUF):
        in_copy(i).start()

    vo_ref[...] = v_ref[...]
    mo_ref[...] = m_ref[...] != 0.0

    for i in range(nchunk):
        in_copy(i).wait()
        out_copy(i).start()
        if i + _NBUF < nchunk:
            out_copy(i).wait()
            in_copy(i + _NBUF).start()

    for i in range(max(nchunk - _NBUF, 0), nchunk):
        out_copy(i).wait()


def kernel(x, v, mask):
    B, H, L = x.shape
    out = pl.pallas_call(
        _trim_passthrough_kernel,
        in_specs=[
            pl.BlockSpec(memory_space=pl.ANY),
            pl.BlockSpec(v.shape, lambda: (0, 0, 0)),
            pl.BlockSpec(mask.shape, lambda: (0, 0, 0)),
        ],
        out_specs=[
            pl.BlockSpec(memory_space=pl.ANY),
            pl.BlockSpec(v.shape, lambda: (0, 0, 0)),
            pl.BlockSpec(mask.shape, lambda: (0, 0, 0)),
        ],
        out_shape=[
            jax.ShapeDtypeStruct(x.shape, x.dtype),
            jax.ShapeDtypeStruct(v.shape, v.dtype),
            jax.ShapeDtypeStruct(mask.shape, jnp.bool_),
        ],
        scratch_shapes=[
            pltpu.VMEM((_NBUF, H, L), x.dtype),
            pltpu.SemaphoreType.DMA((_NBUF,)),
            pltpu.SemaphoreType.DMA((_NBUF,)),
        ],
    )(x, v, mask)
    return (out[0], out[1], out[2])


# speedup vs baseline: 1.6485x; 1.2976x over previous
import jax
import jax.numpy as jnp
from jax.experimental import pallas as pl
from jax.experimental.pallas import tpu as pltpu

_NBUF = 2
_NCHUNK = 4


def _trim_passthrough_kernel(x_hbm, v_ref, m_ref, xo_hbm, vo_ref, mo_ref,
                             buf, sem_in, sem_out):
    nchunk = _NCHUNK
    rows = x_hbm.shape[0] // _NCHUNK

    def in_copy(i):
        return pltpu.make_async_copy(x_hbm.at[pl.ds(i * rows, rows)],
                                     buf.at[i % _NBUF], sem_in.at[i % _NBUF])

    def out_copy(i):
        return pltpu.make_async_copy(buf.at[i % _NBUF],
                                     xo_hbm.at[pl.ds(i * rows, rows)],
                                     sem_out.at[i % _NBUF])

    for i in range(min(_NBUF, nchunk)):
        in_copy(i).start()

    vo_ref[...] = v_ref[...]
    mo_ref[...] = m_ref[...] != 0.0

    for i in range(nchunk):
        in_copy(i).wait()
        out_copy(i).start()
        if i + _NBUF < nchunk:
            out_copy(i).wait()
            in_copy(i + _NBUF).start()

    for i in range(max(nchunk - _NBUF, 0), nchunk):
        out_copy(i).wait()


def kernel(x, v, mask):
    B, H, L = x.shape
    out = pl.pallas_call(
        _trim_passthrough_kernel,
        in_specs=[
            pl.BlockSpec(memory_space=pl.ANY),
            pl.BlockSpec(v.shape, lambda: (0, 0, 0)),
            pl.BlockSpec(mask.shape, lambda: (0, 0, 0)),
        ],
        out_specs=[
            pl.BlockSpec(memory_space=pl.ANY),
            pl.BlockSpec(v.shape, lambda: (0, 0, 0)),
            pl.BlockSpec(mask.shape, lambda: (0, 0, 0)),
        ],
        out_shape=[
            jax.ShapeDtypeStruct(x.shape, x.dtype),
            jax.ShapeDtypeStruct(v.shape, v.dtype),
            jax.ShapeDtypeStruct(mask.shape, jnp.bool_),
        ],
        scratch_shapes=[
            pltpu.VMEM((_NBUF, B // _NCHUNK, H, L), x.dtype),
            pltpu.SemaphoreType.DMA((_NBUF,)),
            pltpu.SemaphoreType.DMA((_NBUF,)),
        ],
    )(x, v, mask)
    return (out[0], out[1], out[2])


# speedup vs baseline: 1.8418x; 1.1173x over previous
import jax
import jax.numpy as jnp
from jax.experimental import pallas as pl
from jax.experimental.pallas import tpu as pltpu

_NBUF = 2
_NCHUNK = 2


def _trim_passthrough_kernel(x_hbm, v_ref, m_ref, xo_hbm, vo_ref, mo_ref,
                             buf, sem_in, sem_out):
    nchunk = _NCHUNK
    rows = x_hbm.shape[0] // _NCHUNK

    def in_copy(i):
        return pltpu.make_async_copy(x_hbm.at[pl.ds(i * rows, rows)],
                                     buf.at[i % _NBUF], sem_in.at[i % _NBUF])

    def out_copy(i):
        return pltpu.make_async_copy(buf.at[i % _NBUF],
                                     xo_hbm.at[pl.ds(i * rows, rows)],
                                     sem_out.at[i % _NBUF])

    for i in range(min(_NBUF, nchunk)):
        in_copy(i).start()

    vo_ref[...] = v_ref[...]
    mo_ref[...] = m_ref[...] != 0.0

    for i in range(nchunk):
        in_copy(i).wait()
        out_copy(i).start()
        if i + _NBUF < nchunk:
            out_copy(i).wait()
            in_copy(i + _NBUF).start()

    for i in range(max(nchunk - _NBUF, 0), nchunk):
        out_copy(i).wait()


def kernel(x, v, mask):
    B, H, L = x.shape
    out = pl.pallas_call(
        _trim_passthrough_kernel,
        in_specs=[
            pl.BlockSpec(memory_space=pl.ANY),
            pl.BlockSpec(v.shape, lambda: (0, 0, 0)),
            pl.BlockSpec(mask.shape, lambda: (0, 0, 0)),
        ],
        out_specs=[
            pl.BlockSpec(memory_space=pl.ANY),
            pl.BlockSpec(v.shape, lambda: (0, 0, 0)),
            pl.BlockSpec(mask.shape, lambda: (0, 0, 0)),
        ],
        out_shape=[
            jax.ShapeDtypeStruct(x.shape, x.dtype),
            jax.ShapeDtypeStruct(v.shape, v.dtype),
            jax.ShapeDtypeStruct(mask.shape, jnp.bool_),
        ],
        scratch_shapes=[
            pltpu.VMEM((_NBUF, B // _NCHUNK, H, L), x.dtype),
            pltpu.SemaphoreType.DMA((_NBUF,)),
            pltpu.SemaphoreType.DMA((_NBUF,)),
        ],
    )(x, v, mask)
    return (out[0], out[1], out[2])
